# baseline (speedup 1.0000x reference)
import functools

import jax
import jax.numpy as jnp
from jax import lax
from jax.experimental import pallas as pl
from jax.experimental.pallas import tpu as pltpu
from jax.experimental.pallas import tpu_sc as plsc

_D = 64
_NC = 2
_NS = 16
_NW = _NC * _NS
_L = 128


@functools.lru_cache(maxsize=None)
def _build_gather(N: int):
    NR = N // _L
    r_per_w = NR // _NW
    ch_r = 8
    ch = ch_r * _L
    n_ch = r_per_w // ch_r
    assert n_ch * ch_r == r_per_w

    mesh = plsc.VectorSubcoreMesh(core_axis_name="c", subcore_axis_name="s")

    @functools.partial(
        pl.kernel,
        mesh=mesh,
        out_type=jax.ShapeDtypeStruct((N, _D), jnp.float32),
        compiler_params=pltpu.CompilerParams(use_tc_tiling_on_sc=False),
        scratch_types=[
            pltpu.VMEM((ch_r, _L), jnp.int32),
            pltpu.VMEM((ch, _D), jnp.float32),
            pltpu.SemaphoreType.DMA,
        ],
    )
    def gather_kernel(ids_hbm, table_hbm, out_hbm, idx_v, rows_v, gsem):
        wid = lax.axis_index("s") * _NC + lax.axis_index("c")
        row_base = wid * r_per_w

        @pl.loop(0, n_ch)
        def _chunk(g):
            roff = row_base + g * ch_r
            pltpu.sync_copy(ids_hbm.at[pl.ds(roff, ch_r)], idx_v)
            copies = [
                pltpu.async_copy(
                    table_hbm.at[idx_v.at[j]],
                    rows_v.at[pl.ds(j * _L, _L)],
                    gsem,
                )
                for j in range(ch_r)
            ]
            for c in copies:
                c.wait()
            pltpu.sync_copy(rows_v, out_hbm.at[pl.ds(roff * _L, ch)])

    return gather_kernel


def kernel(token_ids, embedding):
    b, s = token_ids.shape
    n = b * s
    ids = token_ids.reshape(n // _L, _L).astype(jnp.int32)
    out = _build_gather(n)(ids, embedding)
    return out.reshape(b, s, _D)


# speedup vs baseline: 1.0052x; 1.0052x over previous
import functools

import jax
import jax.numpy as jnp
from jax import lax
from jax.experimental import pallas as pl
from jax.experimental.pallas import tpu as pltpu
from jax.experimental.pallas import tpu_sc as plsc

_D = 64
_NC = 2
_NS = 16
_NW = _NC * _NS
_L = 128


@functools.lru_cache(maxsize=None)
def _build_gather(N: int):
    NR = N // _L
    r_per_w = NR // _NW
    ch_r = 4
    ch = ch_r * _L
    n_ch = r_per_w // ch_r
    assert n_ch * ch_r == r_per_w and n_ch % 2 == 0

    mesh = plsc.VectorSubcoreMesh(core_axis_name="c", subcore_axis_name="s")

    @functools.partial(
        pl.kernel,
        mesh=mesh,
        out_type=jax.ShapeDtypeStruct((N, _D), jnp.float32),
        compiler_params=pltpu.CompilerParams(use_tc_tiling_on_sc=False),
        scratch_types=[
            pltpu.VMEM((r_per_w, _L), jnp.int32),
            pltpu.VMEM((2, ch, _D), jnp.float32),
            pltpu.SemaphoreType.DMA,
            pltpu.SemaphoreType.DMA,
            pltpu.SemaphoreType.DMA,
            pltpu.SemaphoreType.DMA,
        ],
    )
    def gather_kernel(ids_hbm, table_hbm, out_hbm, idx_v, rows_v,
                      gs0, gs1, ss0, ss1):
        wid = lax.axis_index("s") * _NC + lax.axis_index("c")
        row_base = wid * r_per_w
        gsems = (gs0, gs1)
        ssems = (ss0, ss1)

        pltpu.sync_copy(ids_hbm.at[pl.ds(row_base, r_per_w)], idx_v)

        def gather_descs(g, b):
            return [
                pltpu.make_async_copy(
                    table_hbm.at[idx_v.at[g * ch_r + j]],
                    rows_v.at[b].at[pl.ds(j * _L, _L)],
                    gsems[b],
                )
                for j in range(ch_r)
            ]

        def store_desc(g, b):
            return pltpu.make_async_copy(
                rows_v.at[b],
                out_hbm.at[pl.ds((row_base + g * ch_r) * _L, ch)],
                ssems[b],
            )

        def fire_gather(g, b):
            for d in gather_descs(g, b):
                d.start()

        fire_gather(0, 0)
        fire_gather(1, 1)

        @pl.loop(0, n_ch, step=2)
        def _pair(t):
            for b in range(2):
                g = t + b
                for d in gather_descs(g, b):
                    d.wait()
                store_desc(g, b).start()
            for b in range(2):
                g = t + b
                store_desc(g, b).wait()

                @pl.when(g + 2 < n_ch)
                def _():
                    fire_gather(g + 2, b)

    return gather_kernel


def kernel(token_ids, embedding):
    b, s = token_ids.shape
    n = b * s
    ids = token_ids.reshape(n // _L, _L).astype(jnp.int32)
    out = _build_gather(n)(ids, embedding)
    return out.reshape(b, s, _D)
